# Initial kernel scaffold; baseline (speedup 1.0000x reference)
import jax
import jax.numpy as jnp
from jax.experimental import pallas as pl


def kernel(x, adj_t, W_embed, b_embed, W1, W2, W3, W4):
    raise NotImplementedError("write your pallas kernel here")


# speedup vs baseline: 20.0899x; 20.0899x over previous
import functools
import math

import jax
import jax.numpy as jnp
from jax import lax
from jax.experimental import pallas as pl
from jax.experimental.pallas import tpu as pltpu
from jax.experimental.pallas import tpu_sc as plsc

N_NODES = 10000
N_EDGES = 320000
IN_C = 128
HID = 64
ALPHA = 0.1
THETA = 0.5

NC, NS = 2, 16
NW = NC * NS
CH = 120
NCH = 86
PER_W = NCH * CH
E_PAD = NW * PER_W
ACC_ROWS = 10240
ROWS_PER_TILE = ACC_ROWS // NS
DEG_W = 16

_sc_mesh = plsc.VectorSubcoreMesh(core_axis_name="c", subcore_axis_name="s")


def _spmm_body(g_hbm, row_hbm, col_hbm, out_hbm, idx_r, idx_c, buf, acc_sh, sem):
    c = lax.axis_index("c")
    s = lax.axis_index("s")
    w = c * NS + s
    pltpu.sync_copy(row_hbm.at[w], idx_r)
    pltpu.sync_copy(col_hbm.at[w], idx_c)

    z16 = jnp.zeros((16,), jnp.float32)

    def zrow(i, carry):
        for j in range(HID // 16):
            buf[i, pl.ds(j * 16, 16)] = z16
        return carry

    lax.fori_loop(0, CH, zrow, 0)
    base = s * ROWS_PER_TILE
    for k in range(ROWS_PER_TILE // CH):
        pltpu.sync_copy(buf, acc_sh.at[pl.ds(base + k * CH, CH)])
    rem = ROWS_PER_TILE % CH
    if rem:
        pltpu.sync_copy(buf.at[pl.ds(0, rem)],
                        acc_sh.at[pl.ds(base + (ROWS_PER_TILE // CH) * CH, rem)])
    plsc.subcore_barrier()

    def chunk(j, carry):
        pltpu.async_copy(g_hbm.at[idx_r.at[j]], buf, sem).wait()
        pltpu.sync_copy(buf, acc_sh.at[idx_c.at[j]], add=True)
        return carry

    lax.fori_loop(0, NCH, chunk, 0)
    plsc.subcore_barrier()
    pltpu.sync_copy(acc_sh.at[pl.ds(base, ROWS_PER_TILE)],
                    out_hbm.at[c].at[pl.ds(base, ROWS_PER_TILE)])


_spmm = pl.kernel(
    _spmm_body,
    out_type=jax.ShapeDtypeStruct((NC, ACC_ROWS, HID), jnp.float32),
    mesh=_sc_mesh,
    compiler_params=pltpu.CompilerParams(use_tc_tiling_on_sc=False),
    scratch_types=[
        pltpu.VMEM((NCH, CH), jnp.int32),
        pltpu.VMEM((NCH, CH), jnp.int32),
        pltpu.VMEM((CH, HID), jnp.float32),
        pltpu.VMEM_SHARED((ACC_ROWS, HID), jnp.float32),
        pltpu.SemaphoreType.DMA,
    ],
)


def _deg_body(col_hbm, out_hbm, idx_c, buf, acc_sh):
    c = lax.axis_index("c")
    s = lax.axis_index("s")
    w = c * NS + s
    pltpu.sync_copy(col_hbm.at[w], idx_c)

    z16 = jnp.zeros((16,), jnp.float32)

    def zrow(i, carry):
        buf[i, :] = z16
        return carry

    lax.fori_loop(0, CH, zrow, 0)
    base = s * ROWS_PER_TILE
    for k in range(ROWS_PER_TILE // CH):
        pltpu.sync_copy(buf, acc_sh.at[pl.ds(base + k * CH, CH)])
    rem = ROWS_PER_TILE % CH
    if rem:
        pltpu.sync_copy(buf.at[pl.ds(0, rem)],
                        acc_sh.at[pl.ds(base + (ROWS_PER_TILE // CH) * CH, rem)])

    one16 = jnp.ones((16,), jnp.float32)

    def orow(i, carry):
        buf[i, :] = one16
        return carry

    lax.fori_loop(0, CH, orow, 0)
    plsc.subcore_barrier()

    def chunk(j, carry):
        pltpu.sync_copy(buf, acc_sh.at[idx_c.at[j]], add=True)
        return carry

    lax.fori_loop(0, NCH, chunk, 0)
    plsc.subcore_barrier()
    pltpu.sync_copy(acc_sh.at[pl.ds(base, ROWS_PER_TILE)],
                    out_hbm.at[c].at[pl.ds(base, ROWS_PER_TILE)])


_deg = pl.kernel(
    _deg_body,
    out_type=jax.ShapeDtypeStruct((NC, ACC_ROWS, DEG_W), jnp.float32),
    mesh=_sc_mesh,
    compiler_params=pltpu.CompilerParams(use_tc_tiling_on_sc=False),
    scratch_types=[
        pltpu.VMEM((NCH, CH), jnp.int32),
        pltpu.VMEM((CH, DEG_W), jnp.float32),
        pltpu.VMEM_SHARED((ACC_ROWS, DEG_W), jnp.float32),
    ],
)


BR = 1000
GRID = N_NODES // BR


def _embed_body(x_ref, we_ref, b_ref, d0_ref, d1_ref, x0_ref, g_ref, dv_ref):
    x0 = jnp.dot(x_ref[...], we_ref[...], preferred_element_type=jnp.float32)
    x0 = jnp.maximum(x0 + b_ref[...], 0.0)
    deg = d0_ref[:, 0:1] + d1_ref[:, 0:1]
    dv = jnp.broadcast_to(lax.rsqrt(deg), (BR, HID))
    x0_ref[...] = x0
    g_ref[...] = dv * x0
    dv_ref[...] = dv


_embed = pl.pallas_call(
    _embed_body,
    grid=(GRID,),
    in_specs=[
        pl.BlockSpec((BR, IN_C), lambda i: (i, 0)),
        pl.BlockSpec((IN_C, HID), lambda i: (0, 0)),
        pl.BlockSpec((1, HID), lambda i: (0, 0)),
        pl.BlockSpec((BR, DEG_W), lambda i: (i, 0)),
        pl.BlockSpec((BR, DEG_W), lambda i: (i, 0)),
    ],
    out_specs=[
        pl.BlockSpec((BR, HID), lambda i: (i, 0)),
        pl.BlockSpec((BR, HID), lambda i: (i, 0)),
        pl.BlockSpec((BR, HID), lambda i: (i, 0)),
    ],
    out_shape=[
        jax.ShapeDtypeStruct((N_NODES, HID), jnp.float32),
        jax.ShapeDtypeStruct((N_NODES, HID), jnp.float32),
        jax.ShapeDtypeStruct((N_NODES, HID), jnp.float32),
    ],
)


def _layer_body(a0_ref, a1_ref, dv_ref, x0_ref, w_ref, o_ref, *, beta, final):
    dv = dv_ref[...]
    p = dv * (a0_ref[...] + a1_ref[...])
    t = (1.0 - ALPHA) * p + ALPHA * x0_ref[...]
    out = (1.0 - beta) * t + beta * jnp.dot(
        t, w_ref[...], preferred_element_type=jnp.float32)
    if final:
        o_ref[...] = out
    else:
        o_ref[...] = dv * jnp.maximum(out, 0.0)


def _make_layer(beta, final):
    return pl.pallas_call(
        functools.partial(_layer_body, beta=beta, final=final),
        grid=(GRID,),
        in_specs=[
            pl.BlockSpec((BR, HID), lambda i: (i, 0)),
            pl.BlockSpec((BR, HID), lambda i: (i, 0)),
            pl.BlockSpec((BR, HID), lambda i: (i, 0)),
            pl.BlockSpec((BR, HID), lambda i: (i, 0)),
            pl.BlockSpec((HID, HID), lambda i: (0, 0)),
        ],
        out_specs=pl.BlockSpec((BR, HID), lambda i: (i, 0)),
        out_shape=jax.ShapeDtypeStruct((N_NODES, HID), jnp.float32),
    )


_layers = [_make_layer(math.log(THETA / l + 1.0), final=(l == 4))
           for l in range(1, 5)]


def kernel(x, adj_t, W_embed, b_embed, W1, W2, W3, W4):
    row = adj_t[0].astype(jnp.int32)
    col = adj_t[1].astype(jnp.int32)
    loop = jnp.arange(N_NODES, dtype=jnp.int32)
    pad = E_PAD - (N_EDGES + N_NODES)
    r = jnp.concatenate([row, loop, jnp.zeros((pad,), jnp.int32)])
    c = jnp.concatenate([col, loop, jnp.full((pad,), N_NODES, jnp.int32)])
    row3 = r.reshape(NW, NCH, CH)
    col3 = c.reshape(NW, NCH, CH)

    degp = _deg(col3)
    x0, g, dv = _embed(x, W_embed, b_embed.reshape(1, HID),
                       degp[0, :N_NODES], degp[1, :N_NODES])
    for lyr, W in zip(_layers, (W1, W2, W3, W4)):
        acc = _spmm(g, row3, col3)
        g = lyr(acc[0, :N_NODES], acc[1, :N_NODES], dv, x0, W)
    return g


# speedup vs baseline: 22.3373x; 1.1119x over previous
import functools
import math

import jax
import jax.numpy as jnp
from jax import lax
from jax.experimental import pallas as pl
from jax.experimental.pallas import tpu as pltpu
from jax.experimental.pallas import tpu_sc as plsc

N_NODES = 10000
N_EDGES = 320000
IN_C = 128
HID = 64
ALPHA = 0.1
THETA = 0.5

NC, NS = 2, 16
NW = NC * NS
CH = 80
NCH = N_EDGES // (NW * CH)
ROWS_PER_TILE = N_NODES // NS
DEG_W = 16

_sc_mesh = plsc.VectorSubcoreMesh(core_axis_name="c", subcore_axis_name="s")


def _zero_acc_slice(buf, acc_sh, base):
    z16 = jnp.zeros((16,), jnp.float32)
    width = buf.shape[1]

    def zrow(i, carry):
        for j in range(width // 16):
            buf[i, pl.ds(j * 16, 16)] = z16
        return carry

    lax.fori_loop(0, CH, zrow, 0)
    for k in range(ROWS_PER_TILE // CH):
        pltpu.sync_copy(buf, acc_sh.at[pl.ds(base + k * CH, CH)])
    rem = ROWS_PER_TILE % CH
    if rem:
        pltpu.sync_copy(buf.at[pl.ds(0, rem)],
                        acc_sh.at[pl.ds(base + (ROWS_PER_TILE // CH) * CH, rem)])


def _spmm_body(g_hbm, row_hbm, col_hbm, out_hbm,
               idx_r, idx_c, buf_a, buf_b, acc_sh, sem_a, sem_b):
    c = lax.axis_index("c")
    s = lax.axis_index("s")
    w = c * NS + s
    pltpu.sync_copy(row_hbm.at[w], idx_r)
    pltpu.sync_copy(col_hbm.at[w], idx_c)
    base = s * ROWS_PER_TILE
    _zero_acc_slice(buf_a, acc_sh, base)
    plsc.subcore_barrier()

    pltpu.async_copy(g_hbm.at[idx_r.at[0]], buf_a, sem_a)

    def pair(t, carry):
        j = 2 * t
        pltpu.make_async_copy(g_hbm.at[idx_r.at[j]], buf_a, sem_a).wait()
        pltpu.async_copy(g_hbm.at[idx_r.at[j + 1]], buf_b, sem_b)
        pltpu.sync_copy(buf_a, acc_sh.at[idx_c.at[j]], add=True)
        pltpu.make_async_copy(g_hbm.at[idx_r.at[j + 1]], buf_b, sem_b).wait()
        pltpu.async_copy(g_hbm.at[idx_r.at[j + 2]], buf_a, sem_a)
        pltpu.sync_copy(buf_b, acc_sh.at[idx_c.at[j + 1]], add=True)
        return carry

    lax.fori_loop(0, (NCH - 1) // 2, pair, 0)
    pltpu.make_async_copy(g_hbm.at[idx_r.at[NCH - 1]], buf_a, sem_a).wait()
    pltpu.sync_copy(buf_a, acc_sh.at[idx_c.at[NCH - 1]], add=True)

    plsc.subcore_barrier()
    pltpu.sync_copy(acc_sh.at[pl.ds(base, ROWS_PER_TILE)],
                    out_hbm.at[c].at[pl.ds(base, ROWS_PER_TILE)])


_spmm = pl.kernel(
    _spmm_body,
    out_type=jax.ShapeDtypeStruct((NC, N_NODES, HID), jnp.float32),
    mesh=_sc_mesh,
    compiler_params=pltpu.CompilerParams(use_tc_tiling_on_sc=False),
    scratch_types=[
        pltpu.VMEM((NCH, CH), jnp.int32),
        pltpu.VMEM((NCH, CH), jnp.int32),
        pltpu.VMEM((CH, HID), jnp.float32),
        pltpu.VMEM((CH, HID), jnp.float32),
        pltpu.VMEM_SHARED((N_NODES, HID), jnp.float32),
        pltpu.SemaphoreType.DMA,
        pltpu.SemaphoreType.DMA,
    ],
)


def _deg_body(col_hbm, out_hbm, idx_c, buf, acc_sh):
    c = lax.axis_index("c")
    s = lax.axis_index("s")
    w = c * NS + s
    pltpu.sync_copy(col_hbm.at[w], idx_c)
    base = s * ROWS_PER_TILE
    _zero_acc_slice(buf, acc_sh, base)

    one16 = jnp.ones((16,), jnp.float32)

    def orow(i, carry):
        buf[i, :] = one16
        return carry

    lax.fori_loop(0, CH, orow, 0)
    plsc.subcore_barrier()

    def chunk(j, carry):
        pltpu.sync_copy(buf, acc_sh.at[idx_c.at[j]], add=True)
        return carry

    lax.fori_loop(0, NCH, chunk, 0)
    plsc.subcore_barrier()
    pltpu.sync_copy(acc_sh.at[pl.ds(base, ROWS_PER_TILE)],
                    out_hbm.at[c].at[pl.ds(base, ROWS_PER_TILE)])


_deg = pl.kernel(
    _deg_body,
    out_type=jax.ShapeDtypeStruct((NC, N_NODES, DEG_W), jnp.float32),
    mesh=_sc_mesh,
    compiler_params=pltpu.CompilerParams(use_tc_tiling_on_sc=False),
    scratch_types=[
        pltpu.VMEM((NCH, CH), jnp.int32),
        pltpu.VMEM((CH, DEG_W), jnp.float32),
        pltpu.VMEM_SHARED((N_NODES, DEG_W), jnp.float32),
    ],
)


BR = 1000
GRID = N_NODES // BR


def _embed_body(x_ref, we_ref, b_ref, dp_ref, x0_ref, g_ref, dv_ref):
    x0 = jnp.dot(x_ref[...], we_ref[...], preferred_element_type=jnp.float32)
    x0 = jnp.maximum(x0 + b_ref[...], 0.0)
    deg = dp_ref[0, :, 0:1] + dp_ref[1, :, 0:1] + 1.0
    dv = jnp.broadcast_to(lax.rsqrt(deg), (BR, HID))
    x0_ref[...] = x0
    g_ref[...] = dv * x0
    dv_ref[...] = dv


_embed = pl.pallas_call(
    _embed_body,
    grid=(GRID,),
    in_specs=[
        pl.BlockSpec((BR, IN_C), lambda i: (i, 0)),
        pl.BlockSpec((IN_C, HID), lambda i: (0, 0)),
        pl.BlockSpec((1, HID), lambda i: (0, 0)),
        pl.BlockSpec((NC, BR, DEG_W), lambda i: (0, i, 0)),
    ],
    out_specs=[
        pl.BlockSpec((BR, HID), lambda i: (i, 0)),
        pl.BlockSpec((BR, HID), lambda i: (i, 0)),
        pl.BlockSpec((BR, HID), lambda i: (i, 0)),
    ],
    out_shape=[
        jax.ShapeDtypeStruct((N_NODES, HID), jnp.float32),
        jax.ShapeDtypeStruct((N_NODES, HID), jnp.float32),
        jax.ShapeDtypeStruct((N_NODES, HID), jnp.float32),
    ],
)


def _layer_body(acc_ref, g_ref, dv_ref, x0_ref, w_ref, o_ref, *, beta, final):
    dv = dv_ref[...]
    p = dv * (acc_ref[0] + acc_ref[1] + g_ref[...])
    t = (1.0 - ALPHA) * p + ALPHA * x0_ref[...]
    out = (1.0 - beta) * t + beta * jnp.dot(
        t, w_ref[...], preferred_element_type=jnp.float32)
    if final:
        o_ref[...] = out
    else:
        o_ref[...] = dv * jnp.maximum(out, 0.0)


def _make_layer(beta, final):
    return pl.pallas_call(
        functools.partial(_layer_body, beta=beta, final=final),
        grid=(GRID,),
        in_specs=[
            pl.BlockSpec((NC, BR, HID), lambda i: (0, i, 0)),
            pl.BlockSpec((BR, HID), lambda i: (i, 0)),
            pl.BlockSpec((BR, HID), lambda i: (i, 0)),
            pl.BlockSpec((BR, HID), lambda i: (i, 0)),
            pl.BlockSpec((HID, HID), lambda i: (0, 0)),
        ],
        out_specs=pl.BlockSpec((BR, HID), lambda i: (i, 0)),
        out_shape=jax.ShapeDtypeStruct((N_NODES, HID), jnp.float32),
    )


_layers = [_make_layer(math.log(THETA / l + 1.0), final=(l == 4))
           for l in range(1, 5)]


def kernel(x, adj_t, W_embed, b_embed, W1, W2, W3, W4):
    adj = adj_t.astype(jnp.int32)
    row3 = adj[0].reshape(NW, NCH, CH)
    col3 = adj[1].reshape(NW, NCH, CH)

    degp = _deg(col3)
    x0, g, dv = _embed(x, W_embed, b_embed.reshape(1, HID), degp)
    for lyr, W in zip(_layers, (W1, W2, W3, W4)):
        acc = _spmm(g, row3, col3)
        g = lyr(acc, g, dv, x0, W)
    return g


# speedup vs baseline: 34.7719x; 1.5567x over previous
import functools
import math

import jax
import jax.numpy as jnp
from jax import lax
from jax.experimental import pallas as pl
from jax.experimental.pallas import tpu as pltpu
from jax.experimental.pallas import tpu_sc as plsc

N_NODES = 10000
N_EDGES = 320000
IN_C = 128
HID = 64
ALPHA = 0.1
THETA = 0.5

NC, NS = 2, 16
NW = NC * NS
CH = 80
NCH = N_EDGES // (NW * CH)
ROWS_PER_TILE = N_NODES // NS
DEG_W = 16

_sc_mesh = plsc.VectorSubcoreMesh(core_axis_name="c", subcore_axis_name="s")


def _zero_acc_slice(buf, acc_sh, base):
    z16 = jnp.zeros((16,), jnp.float32)
    width = buf.shape[1]

    def zrow(i, carry):
        for j in range(width // 16):
            buf[i, pl.ds(j * 16, 16)] = z16
        return carry

    lax.fori_loop(0, CH, zrow, 0)
    for k in range(ROWS_PER_TILE // CH):
        pltpu.sync_copy(buf, acc_sh.at[pl.ds(base + k * CH, CH)])
    rem = ROWS_PER_TILE % CH
    if rem:
        pltpu.sync_copy(buf.at[pl.ds(0, rem)],
                        acc_sh.at[pl.ds(base + (ROWS_PER_TILE // CH) * CH, rem)])


NSLOT = 4


def _spmm_body(g_hbm, row_hbm, col_hbm, out_hbm,
               idx_r, idx_c, b0, b1, b2, b3, acc_sh,
               g0, g1, g2, g3, s0, s1, s2, s3):
    bufs = (b0, b1, b2, b3)
    gsem = (g0, g1, g2, g3)
    ssem = (s0, s1, s2, s3)
    c = lax.axis_index("c")
    s = lax.axis_index("s")
    w = c * NS + s
    pltpu.sync_copy(row_hbm.at[w], idx_r)
    pltpu.sync_copy(col_hbm.at[w], idx_c)
    base = s * ROWS_PER_TILE
    _zero_acc_slice(bufs[0], acc_sh, base)
    plsc.subcore_barrier()

    def _gather(j, b):
        pltpu.async_copy(g_hbm.at[idx_r.at[j]], bufs[b], gsem[b])

    def _wait_gather(j, b):
        pltpu.make_async_copy(g_hbm.at[idx_r.at[j]], bufs[b], gsem[b]).wait()

    def _scatter(j, b):
        pltpu.async_copy(bufs[b], acc_sh.at[idx_c.at[j]], ssem[b], add=True)

    def _wait_scatter(b):
        pltpu.make_async_copy(bufs[b], acc_sh.at[idx_c.at[0]], ssem[b]).wait()

    _gather(0, 0)
    _gather(1, 1)

    def group(t, carry):
        j0 = 4 * t
        for b in range(4):
            j = j0 + b
            sb = (b + 2) % 4

            @pl.when(j >= 2)
            def _():
                _wait_scatter(sb)

            @pl.when(j + 2 < NCH)
            def _():
                _gather(j + 2, sb)

            _wait_gather(j, b)
            _scatter(j, b)
        return carry

    lax.fori_loop(0, NCH // 4, group, 0)
    _wait_gather(NCH - 1, 0)
    _scatter(NCH - 1, 0)
    for b in (0, 2, 3):
        _wait_scatter(b)

    plsc.subcore_barrier()
    pltpu.sync_copy(acc_sh.at[pl.ds(base, ROWS_PER_TILE)],
                    out_hbm.at[c].at[pl.ds(base, ROWS_PER_TILE)])


_spmm = pl.kernel(
    _spmm_body,
    out_type=jax.ShapeDtypeStruct((NC, N_NODES, HID), jnp.float32),
    mesh=_sc_mesh,
    compiler_params=pltpu.CompilerParams(use_tc_tiling_on_sc=False),
    scratch_types=[
        pltpu.VMEM((NCH, CH), jnp.int32),
        pltpu.VMEM((NCH, CH), jnp.int32),
        pltpu.VMEM((CH, HID), jnp.float32),
        pltpu.VMEM((CH, HID), jnp.float32),
        pltpu.VMEM((CH, HID), jnp.float32),
        pltpu.VMEM((CH, HID), jnp.float32),
        pltpu.VMEM_SHARED((N_NODES, HID), jnp.float32),
        pltpu.SemaphoreType.DMA,
        pltpu.SemaphoreType.DMA,
        pltpu.SemaphoreType.DMA,
        pltpu.SemaphoreType.DMA,
        pltpu.SemaphoreType.DMA,
        pltpu.SemaphoreType.DMA,
        pltpu.SemaphoreType.DMA,
        pltpu.SemaphoreType.DMA,
    ],
)


def _deg_body(col_hbm, out_hbm, idx_c, buf, acc_sh):
    c = lax.axis_index("c")
    s = lax.axis_index("s")
    w = c * NS + s
    pltpu.sync_copy(col_hbm.at[w], idx_c)
    base = s * ROWS_PER_TILE
    _zero_acc_slice(buf, acc_sh, base)

    one16 = jnp.ones((16,), jnp.float32)

    def orow(i, carry):
        buf[i, :] = one16
        return carry

    lax.fori_loop(0, CH, orow, 0)
    plsc.subcore_barrier()

    def chunk(j, carry):
        pltpu.sync_copy(buf, acc_sh.at[idx_c.at[j]], add=True)
        return carry

    lax.fori_loop(0, NCH, chunk, 0)
    plsc.subcore_barrier()
    pltpu.sync_copy(acc_sh.at[pl.ds(base, ROWS_PER_TILE)],
                    out_hbm.at[c].at[pl.ds(base, ROWS_PER_TILE)])


_deg = pl.kernel(
    _deg_body,
    out_type=jax.ShapeDtypeStruct((NC, N_NODES, DEG_W), jnp.float32),
    mesh=_sc_mesh,
    compiler_params=pltpu.CompilerParams(use_tc_tiling_on_sc=False),
    scratch_types=[
        pltpu.VMEM((NCH, CH), jnp.int32),
        pltpu.VMEM((CH, DEG_W), jnp.float32),
        pltpu.VMEM_SHARED((N_NODES, DEG_W), jnp.float32),
    ],
)


BR = 1000
GRID = N_NODES // BR


def _embed_body(x_ref, we_ref, b_ref, x0_ref):
    x0 = jnp.dot(x_ref[...], we_ref[...], preferred_element_type=jnp.float32)
    x0_ref[...] = jnp.maximum(x0 + b_ref[...], 0.0)


_embed = pl.pallas_call(
    _embed_body,
    grid=(GRID,),
    in_specs=[
        pl.BlockSpec((BR, IN_C), lambda i: (i, 0)),
        pl.BlockSpec((IN_C, HID), lambda i: (0, 0)),
        pl.BlockSpec((1, HID), lambda i: (0, 0)),
    ],
    out_specs=pl.BlockSpec((BR, HID), lambda i: (i, 0)),
    out_shape=jax.ShapeDtypeStruct((N_NODES, HID), jnp.float32),
)


def _scale_body(dp_ref, x0_ref, g_ref, dv_ref):
    deg = dp_ref[0, :, 0:1] + dp_ref[1, :, 0:1] + 1.0
    dv = jnp.broadcast_to(lax.rsqrt(deg), (BR, HID))
    g_ref[...] = dv * x0_ref[...]
    dv_ref[...] = dv


_scale = pl.pallas_call(
    _scale_body,
    grid=(GRID,),
    in_specs=[
        pl.BlockSpec((NC, BR, DEG_W), lambda i: (0, i, 0)),
        pl.BlockSpec((BR, HID), lambda i: (i, 0)),
    ],
    out_specs=[
        pl.BlockSpec((BR, HID), lambda i: (i, 0)),
        pl.BlockSpec((BR, HID), lambda i: (i, 0)),
    ],
    out_shape=[
        jax.ShapeDtypeStruct((N_NODES, HID), jnp.float32),
        jax.ShapeDtypeStruct((N_NODES, HID), jnp.float32),
    ],
)


def _layer_body(acc_ref, g_ref, dv_ref, x0_ref, w_ref, o_ref, *, beta, final):
    dv = dv_ref[...]
    p = dv * (acc_ref[0] + acc_ref[1] + g_ref[...])
    t = (1.0 - ALPHA) * p + ALPHA * x0_ref[...]
    out = (1.0 - beta) * t + beta * jnp.dot(
        t, w_ref[...], preferred_element_type=jnp.float32)
    if final:
        o_ref[...] = out
    else:
        o_ref[...] = dv * jnp.maximum(out, 0.0)


def _make_layer(beta, final):
    return pl.pallas_call(
        functools.partial(_layer_body, beta=beta, final=final),
        grid=(GRID,),
        in_specs=[
            pl.BlockSpec((NC, BR, HID), lambda i: (0, i, 0)),
            pl.BlockSpec((BR, HID), lambda i: (i, 0)),
            pl.BlockSpec((BR, HID), lambda i: (i, 0)),
            pl.BlockSpec((BR, HID), lambda i: (i, 0)),
            pl.BlockSpec((HID, HID), lambda i: (0, 0)),
        ],
        out_specs=pl.BlockSpec((BR, HID), lambda i: (i, 0)),
        out_shape=jax.ShapeDtypeStruct((N_NODES, HID), jnp.float32),
    )


_layers = [_make_layer(math.log(THETA / l + 1.0), final=(l == 4))
           for l in range(1, 5)]


def kernel(x, adj_t, W_embed, b_embed, W1, W2, W3, W4):
    adj = adj_t.astype(jnp.int32)
    row3 = adj[0].reshape(NW, NCH, CH)
    col3 = adj[1].reshape(NW, NCH, CH)

    degp = _deg(col3)
    x0 = _embed(x, W_embed, b_embed.reshape(1, HID))
    g, dv = _scale(degp, x0)
    for lyr, W in zip(_layers, (W1, W2, W3, W4)):
        acc = _spmm(g, row3, col3)
        g = lyr(acc, g, dv, x0, W)
    return g
